# Initial kernel scaffold; baseline (speedup 1.0000x reference)
import jax
import jax.numpy as jnp
from jax.experimental import pallas as pl


def kernel(node_features, neighbours, W1, a1, W2, a2, ln_gamma, ln_beta, lin1_W, lin1_b, lin2_W, lin2_b, lin3_W, lin3_b):
    raise NotImplementedError("write your pallas kernel here")


# speedup vs baseline: 35.1241x; 35.1241x over previous
import jax
import jax.numpy as jnp
from jax import lax
from jax.experimental import pallas as pl
from jax.experimental.pallas import tpu as pltpu
from jax.experimental.pallas import tpu_sc as plsc

_N = 10000
_NP = 10240
_DEG = 32
_D = 128
_H = 8
_NB = 4
_ROWS = _NB * _DEG
_NTILES = 32
_TBLK = _NP // (_NB * _NTILES)
_f32 = jnp.float32


def _logits_body(x_ref, bs_ref, bd_ref, es_ref, ed_ref):
    xb = x_ref[:]
    es_ref[:] = jnp.dot(xb, bs_ref[:], preferred_element_type=_f32)
    ed_ref[:] = jnp.dot(xb, bd_ref[:], preferred_element_type=_f32)


def _tc_logits1(x, bsrc, bdst):
    R = 2048
    return pl.pallas_call(
        _logits_body,
        grid=(_NP // R,),
        in_specs=[
            pl.BlockSpec((R, _D), lambda i: (i, 0)),
            pl.BlockSpec((_D, _H), lambda i: (0, 0)),
            pl.BlockSpec((_D, _H), lambda i: (0, 0)),
        ],
        out_specs=[
            pl.BlockSpec((R, _H), lambda i: (i, 0)),
            pl.BlockSpec((R, _H), lambda i: (i, 0)),
        ],
        out_shape=[
            jax.ShapeDtypeStruct((_NP, _H), _f32),
            jax.ShapeDtypeStruct((_NP, _H), _f32),
        ],
    )(x, bsrc, bdst)


def _fuse2_body(s_ref, wc_ref, bs_ref, bd_ref, e1_ref, es_ref, ed_ref):
    agg = jnp.dot(s_ref[:], wc_ref[:], preferred_element_type=_f32)
    z = agg * (1.0 / _H)
    e1 = jnp.where(z > 0, z, jnp.exp(z) - 1.0)
    e1_ref[:] = e1
    es_ref[:] = jnp.dot(e1, bs_ref[:], preferred_element_type=_f32)
    ed_ref[:] = jnp.dot(e1, bd_ref[:], preferred_element_type=_f32)


def _tc_fuse2(s, wcat, bsrc, bdst):
    R = 1024
    return pl.pallas_call(
        _fuse2_body,
        grid=(_NP // R,),
        in_specs=[
            pl.BlockSpec((R, _H * _D), lambda i: (i, 0)),
            pl.BlockSpec((_H * _D, _D), lambda i: (0, 0)),
            pl.BlockSpec((_D, _H), lambda i: (0, 0)),
            pl.BlockSpec((_D, _H), lambda i: (0, 0)),
        ],
        out_specs=[
            pl.BlockSpec((R, _D), lambda i: (i, 0)),
            pl.BlockSpec((R, _H), lambda i: (i, 0)),
            pl.BlockSpec((R, _H), lambda i: (i, 0)),
        ],
        out_shape=[
            jax.ShapeDtypeStruct((_NP, _D), _f32),
            jax.ShapeDtypeStruct((_NP, _H), _f32),
            jax.ShapeDtypeStruct((_NP, _H), _f32),
        ],
    )(s, wcat, bsrc, bdst)


def _head_body(s_ref, wc_ref, gam_ref, bet_ref, w1_ref, b1_ref, w2_ref,
               b2_ref, w3_ref, b3_ref, out_ref, acc_ref):
    i = pl.program_id(0)
    R = s_ref.shape[0]
    agg = jnp.dot(s_ref[:], wc_ref[:], preferred_element_type=_f32)
    z = agg * (1.0 / _H)
    z = jnp.where(z > 0, z, jnp.exp(z) - 1.0)
    rowid = i * R + lax.broadcasted_iota(jnp.int32, (R, _D), 0)
    z = jnp.where(rowid < _N, z, 0.0)
    part = jnp.sum(z, axis=0, keepdims=True)

    @pl.when(i == 0)
    def _():
        acc_ref[:] = part

    @pl.when(i > 0)
    def _():
        acc_ref[:] = acc_ref[:] + part

    @pl.when(i == pl.num_programs(0) - 1)
    def _():
        g = acc_ref[:] * (1.0 / _N)
        mu = jnp.mean(g, axis=1, keepdims=True)
        var = jnp.mean((g - mu) ** 2, axis=1, keepdims=True)
        y = (g - mu) * lax.rsqrt(var + 1e-5) * gam_ref[:] + bet_ref[:]
        h1 = jnp.dot(y, w1_ref[:], preferred_element_type=_f32) + b1_ref[:]
        h1 = jnp.where(h1 > 0, h1, 0.01 * h1)
        h2 = jnp.dot(h1, w2_ref[:], preferred_element_type=_f32) + b2_ref[:]
        h2 = jnp.where(h2 > 0, h2, 0.01 * h2)
        h3 = jnp.dot(h2, w3_ref[:], preferred_element_type=_f32) + b3_ref[:]
        out_ref[:] = jnp.maximum(h3, 0.0)


def _tc_head(s, wcat, gam, bet, w1t, b1, w2t, b2, w3t, b3):
    R = 1024
    full = lambda shape: pl.BlockSpec(shape, lambda i: (0,) * len(shape))
    return pl.pallas_call(
        _head_body,
        grid=(_NP // R,),
        in_specs=[
            pl.BlockSpec((R, _H * _D), lambda i: (i, 0)),
            full((_H * _D, _D)),
            full((1, _D)), full((1, _D)),
            full((_D, 64)), full((1, 64)),
            full((64, 16)), full((1, 16)),
            full((16, 16)), full((1, 16)),
        ],
        out_specs=full((1, 16)),
        out_shape=jax.ShapeDtypeStruct((1, 16), _f32),
        scratch_shapes=[pltpu.VMEM((1, _D), _f32)],
    )(s, wcat, gam, bet, w1t, b1, w2t, b2, w3t, b3)


def _sc_body(table, esrc, edst, nbrs, out, idx_v, es_v, edst_v, xg_v, s_v,
             alpha_v, sem_i, sem_e, sem_g0, sem_g1):
    sem_g = (sem_g0, sem_g1)
    cid = lax.axis_index("c")
    sid = lax.axis_index("s")
    wid = sid * 2 + cid
    base = wid * _TBLK
    lanes_lo = lax.iota(jnp.int32, 16) < 8
    col8 = lax.rem(lax.iota(jnp.int32, 16), 8)

    def node_base(t):
        return (base + t) * _NB

    def fire_idx(t, b):
        nb = node_base(t)
        pltpu.make_async_copy(
            nbrs.at[pl.ds(nb * _DEG, _ROWS)], idx_v.at[b], sem_i).start()
        pltpu.make_async_copy(
            esrc.at[pl.ds(nb * _H, _NB * _H)],
            es_v.at[pl.ds(b * _NB * _H, _NB * _H)], sem_e).start()

    def wait_idx():
        pltpu.make_async_copy(
            nbrs.at[pl.ds(0, _ROWS)], idx_v.at[0], sem_i).wait()
        pltpu.make_async_copy(
            esrc.at[pl.ds(0, _NB * _H)],
            es_v.at[pl.ds(0, _NB * _H)], sem_e).wait()

    def fire_gather(b):
        pltpu.make_async_copy(
            table.at[idx_v.at[b]], xg_v.at[b], sem_g[b]).start()

    def wait_gather(b):
        pltpu.make_async_copy(
            table.at[idx_v.at[b]], xg_v.at[b], sem_g[b]).wait()

    pltpu.sync_copy(edst, edst_v)

    fire_idx(0, 0)
    wait_idx()
    fire_gather(0)
    fire_idx(jnp.minimum(1, _TBLK - 1), 1)

    def softmax_part(cur):
        for p in range(_NB // 2):
            es = es_v[pl.ds(cur * _NB * _H + p * 16, 16)]
            nchunk = [idx_v[cur, pl.ds(p * 2 * _DEG + q * 16, 16)]
                      for q in range(4)]
            evs = []
            for d in range(_DEG):
                v = nchunk[d // 8]
                n0 = v[(2 * d) % 16]
                n1 = v[(2 * d + 1) % 16]
                row = jnp.where(lanes_lo, n0, n1)
                ed = plsc.load_gather(edst_v, [row * 8 + col8])
                e = es + ed
                evs.append(jnp.where(e > 0, e, 0.01 * e))
            m = evs[0]
            for d in range(1, _DEG):
                m = jnp.maximum(m, evs[d])
            ps = [jnp.exp(e - m) for e in evs]
            ssum = ps[0]
            for d in range(1, _DEG):
                ssum = ssum + ps[d]
            rinv = 1.0 / ssum
            for d in range(_DEG):
                alpha_v[p * _DEG + d, :] = ps[d] * rinv

    def accum_part(tt, cur):
        nb = node_base(tt)
        for p in range(_NB // 2):
            for j in range(2):
                for half in range(2):
                    def body(d, accs, _p=p, _j=j, _h=half):
                        r = _p * 2 * _DEG + 2 * d + _j
                        xs = [xg_v[cur, r, pl.ds(_h * 64 + l * 16, 16)]
                              for l in range(4)]
                        av = alpha_v[_p * _DEG + d, :]
                        new = list(accs)
                        for k in range(_H):
                            al = av[_j * 8 + k]
                            for l in range(4):
                                new[k * 4 + l] = new[k * 4 + l] + al * xs[l]
                        return tuple(new)
                    zero = jnp.zeros((16,), _f32)
                    accs = lax.fori_loop(0, _DEG, body, (zero,) * 32)
                    srow = (p * 2 + j) * (_H * _D)
                    for k in range(_H):
                        for l in range(4):
                            s_v[pl.ds(srow + k * _D + half * 64 + l * 16,
                                      16)] = accs[k * 4 + l]
        pltpu.sync_copy(s_v, out.at[pl.ds(nb * (_H * _D), _NB * _H * _D)])

    def outer(o, carry):
        for b2 in range(2):
            tt = o * 2 + b2
            cur, nxt = b2, 1 - b2
            wait_idx()
            fire_gather(nxt)
            wait_gather(cur)
            softmax_part(cur)
            fire_idx(jnp.minimum(tt + 2, _TBLK - 1), cur)
            accum_part(tt, cur)
        return carry

    lax.fori_loop(0, _TBLK // 2, outer, 0)
    wait_idx()
    wait_gather(0)


_sc_agg_built = None


def _sc_agg(table, esrc_flat, edst_flat, nbrs_flat):
    global _sc_agg_built
    if _sc_agg_built is None:
        mesh = plsc.VectorSubcoreMesh(core_axis_name="c", subcore_axis_name="s")
        _sc_agg_built = pl.kernel(
            _sc_body,
            out_type=jax.ShapeDtypeStruct((_NP * _H * _D,), _f32),
            mesh=mesh,
            compiler_params=pltpu.CompilerParams(needs_layout_passes=False),
            scratch_types=[
                pltpu.VMEM((2, _ROWS), jnp.int32),
                pltpu.VMEM((2 * _NB * _H,), _f32),
                pltpu.VMEM((_NP * _H,), _f32),
                pltpu.VMEM((2, _ROWS, _D), _f32),
                pltpu.VMEM((_NB * _H * _D,), _f32),
                pltpu.VMEM(((_NB // 2) * _DEG, 16), _f32),
                pltpu.SemaphoreType.DMA,
                pltpu.SemaphoreType.DMA,
                pltpu.SemaphoreType.DMA,
                pltpu.SemaphoreType.DMA,
            ],
        )
    return _sc_agg_built(table, esrc_flat, edst_flat, nbrs_flat)


def kernel(node_features, neighbours, W1, a1, W2, a2, ln_gamma, ln_beta,
           lin1_W, lin1_b, lin2_W, lin2_b, lin3_W, lin3_b):
    pad = _NP - _N
    x = jnp.pad(node_features, ((0, pad), (0, 0)))
    nbrs = jnp.pad(neighbours.astype(jnp.int32), ((0, pad), (0, 0)))
    nbp = nbrs.reshape(_NP // 2, 2, _DEG).transpose(0, 2, 1).reshape(-1)

    b1s = jnp.einsum('khc,kh->ck', W1, a1[:, :_D])
    b1d = jnp.einsum('khc,kh->ck', W1, a1[:, _D:])
    b2s = jnp.einsum('khc,kh->ck', W2, a2[:, :_D])
    b2d = jnp.einsum('khc,kh->ck', W2, a2[:, _D:])
    w1c = W1.transpose(0, 2, 1).reshape(_H * _D, _D)
    w2c = W2.transpose(0, 2, 1).reshape(_H * _D, _D)

    es1, ed1 = _tc_logits1(x, b1s, b1d)
    s1 = _sc_agg(x, es1.reshape(-1), ed1.reshape(-1), nbp).reshape(_NP, -1)
    e1, es2, ed2 = _tc_fuse2(s1, w1c, b2s, b2d)
    s2 = _sc_agg(e1, es2.reshape(-1), ed2.reshape(-1), nbp).reshape(_NP, -1)
    out = _tc_head(s2, w2c,
                   ln_gamma.reshape(1, -1), ln_beta.reshape(1, -1),
                   lin1_W.T, lin1_b.reshape(1, -1),
                   lin2_W.T, lin2_b.reshape(1, -1),
                   lin3_W.T, lin3_b.reshape(1, -1))
    return out.reshape(16)


# speedup vs baseline: 35.1941x; 1.0020x over previous
import jax
import jax.numpy as jnp
from jax import lax
from jax.experimental import pallas as pl
from jax.experimental.pallas import tpu as pltpu
from jax.experimental.pallas import tpu_sc as plsc

_N = 10000
_NP = 10240
_DEG = 32
_D = 128
_H = 8
_NB = 4
_ROWS = _NB * _DEG
_NTILES = 32
_TBLK = _NP // (_NB * _NTILES)
_f32 = jnp.float32


def _logits_body(x_ref, bs_ref, bd_ref, es_ref, ed_ref):
    xb = x_ref[:]
    es_ref[:] = jnp.dot(xb, bs_ref[:], preferred_element_type=_f32)
    ed_ref[:] = jnp.dot(xb, bd_ref[:], preferred_element_type=_f32)


def _tc_logits1(x, bsrc, bdst):
    R = 2048
    return pl.pallas_call(
        _logits_body,
        grid=(_NP // R,),
        in_specs=[
            pl.BlockSpec((R, _D), lambda i: (i, 0)),
            pl.BlockSpec((_D, _H), lambda i: (0, 0)),
            pl.BlockSpec((_D, _H), lambda i: (0, 0)),
        ],
        out_specs=[
            pl.BlockSpec((R, _H), lambda i: (i, 0)),
            pl.BlockSpec((R, _H), lambda i: (i, 0)),
        ],
        out_shape=[
            jax.ShapeDtypeStruct((_NP, _H), _f32),
            jax.ShapeDtypeStruct((_NP, _H), _f32),
        ],
    )(x, bsrc, bdst)


def _fuse2_body(s_ref, wc_ref, bs_ref, bd_ref, e1_ref, es_ref, ed_ref):
    agg = jnp.dot(s_ref[:], wc_ref[:], preferred_element_type=_f32)
    z = agg * (1.0 / _H)
    e1 = jnp.where(z > 0, z, jnp.exp(z) - 1.0)
    e1_ref[:] = e1
    es_ref[:] = jnp.dot(e1, bs_ref[:], preferred_element_type=_f32)
    ed_ref[:] = jnp.dot(e1, bd_ref[:], preferred_element_type=_f32)


def _tc_fuse2(s, wcat, bsrc, bdst):
    R = 1024
    return pl.pallas_call(
        _fuse2_body,
        grid=(_NP // R,),
        in_specs=[
            pl.BlockSpec((R, _H * _D), lambda i: (i, 0)),
            pl.BlockSpec((_H * _D, _D), lambda i: (0, 0)),
            pl.BlockSpec((_D, _H), lambda i: (0, 0)),
            pl.BlockSpec((_D, _H), lambda i: (0, 0)),
        ],
        out_specs=[
            pl.BlockSpec((R, _D), lambda i: (i, 0)),
            pl.BlockSpec((R, _H), lambda i: (i, 0)),
            pl.BlockSpec((R, _H), lambda i: (i, 0)),
        ],
        out_shape=[
            jax.ShapeDtypeStruct((_NP, _D), _f32),
            jax.ShapeDtypeStruct((_NP, _H), _f32),
            jax.ShapeDtypeStruct((_NP, _H), _f32),
        ],
    )(s, wcat, bsrc, bdst)


def _head_body(s_ref, wc_ref, gam_ref, bet_ref, w1_ref, b1_ref, w2_ref,
               b2_ref, w3_ref, b3_ref, out_ref, acc_ref):
    i = pl.program_id(0)
    R = s_ref.shape[0]
    agg = jnp.dot(s_ref[:], wc_ref[:], preferred_element_type=_f32)
    z = agg * (1.0 / _H)
    z = jnp.where(z > 0, z, jnp.exp(z) - 1.0)
    rowid = i * R + lax.broadcasted_iota(jnp.int32, (R, _D), 0)
    z = jnp.where(rowid < _N, z, 0.0)
    part = jnp.sum(z, axis=0, keepdims=True)

    @pl.when(i == 0)
    def _():
        acc_ref[:] = part

    @pl.when(i > 0)
    def _():
        acc_ref[:] = acc_ref[:] + part

    @pl.when(i == pl.num_programs(0) - 1)
    def _():
        g = acc_ref[:] * (1.0 / _N)
        mu = jnp.mean(g, axis=1, keepdims=True)
        var = jnp.mean((g - mu) ** 2, axis=1, keepdims=True)
        y = (g - mu) * lax.rsqrt(var + 1e-5) * gam_ref[:] + bet_ref[:]
        h1 = jnp.dot(y, w1_ref[:], preferred_element_type=_f32) + b1_ref[:]
        h1 = jnp.where(h1 > 0, h1, 0.01 * h1)
        h2 = jnp.dot(h1, w2_ref[:], preferred_element_type=_f32) + b2_ref[:]
        h2 = jnp.where(h2 > 0, h2, 0.01 * h2)
        h3 = jnp.dot(h2, w3_ref[:], preferred_element_type=_f32) + b3_ref[:]
        out_ref[:] = jnp.maximum(h3, 0.0)


def _tc_head(s, wcat, gam, bet, w1t, b1, w2t, b2, w3t, b3):
    R = 1024
    full = lambda shape: pl.BlockSpec(shape, lambda i: (0,) * len(shape))
    return pl.pallas_call(
        _head_body,
        grid=(_NP // R,),
        in_specs=[
            pl.BlockSpec((R, _H * _D), lambda i: (i, 0)),
            full((_H * _D, _D)),
            full((1, _D)), full((1, _D)),
            full((_D, 64)), full((1, 64)),
            full((64, 16)), full((1, 16)),
            full((16, 16)), full((1, 16)),
        ],
        out_specs=full((1, 16)),
        out_shape=jax.ShapeDtypeStruct((1, 16), _f32),
        scratch_shapes=[pltpu.VMEM((1, _D), _f32)],
    )(s, wcat, gam, bet, w1t, b1, w2t, b2, w3t, b3)


def _sc_body(table, esrc, edst, nbrs, out, idx_v, es_v, edst_v, xg_v, s_v,
             alpha_v, sem_i, sem_e, sem_g0, sem_g1):
    sem_g = (sem_g0, sem_g1)
    cid = lax.axis_index("c")
    sid = lax.axis_index("s")
    wid = sid * 2 + cid
    base = wid * _TBLK
    lanes_lo = lax.iota(jnp.int32, 16) < 8
    col8 = lax.rem(lax.iota(jnp.int32, 16), 8)

    def node_base(t):
        return (base + t) * _NB

    def fire_idx(t, b):
        nb = node_base(t)
        pltpu.make_async_copy(
            nbrs.at[pl.ds(nb * _DEG, _ROWS)], idx_v.at[b], sem_i).start()
        pltpu.make_async_copy(
            esrc.at[pl.ds(nb * _H, _NB * _H)],
            es_v.at[pl.ds(b * _NB * _H, _NB * _H)], sem_e).start()

    def wait_idx():
        pltpu.make_async_copy(
            nbrs.at[pl.ds(0, _ROWS)], idx_v.at[0], sem_i).wait()
        pltpu.make_async_copy(
            esrc.at[pl.ds(0, _NB * _H)],
            es_v.at[pl.ds(0, _NB * _H)], sem_e).wait()

    def fire_gather(b):
        pltpu.make_async_copy(
            table.at[idx_v.at[b]], xg_v.at[b], sem_g[b]).start()

    def wait_gather(b):
        pltpu.make_async_copy(
            table.at[idx_v.at[b]], xg_v.at[b], sem_g[b]).wait()

    pltpu.sync_copy(edst, edst_v)

    fire_idx(0, 0)
    wait_idx()
    fire_gather(0)
    fire_idx(jnp.minimum(1, _TBLK - 1), 1)

    def softmax_part(cur):
        for p in range(_NB // 2):
            es = es_v[pl.ds(cur * _NB * _H + p * 16, 16)]
            nchunk = [idx_v[cur, pl.ds(p * 2 * _DEG + q * 16, 16)]
                      for q in range(4)]
            evs = []
            for d in range(_DEG):
                v = nchunk[d // 8]
                n0 = v[(2 * d) % 16]
                n1 = v[(2 * d + 1) % 16]
                row = jnp.where(lanes_lo, n0, n1)
                ed = plsc.load_gather(edst_v, [row * 8 + col8])
                e = es + ed
                evs.append(jnp.where(e > 0, e, 0.01 * e))
            m = evs[0]
            for d in range(1, _DEG):
                m = jnp.maximum(m, evs[d])
            ps = [jnp.exp(e - m) for e in evs]
            ssum = ps[0]
            for d in range(1, _DEG):
                ssum = ssum + ps[d]
            rinv = 1.0 / ssum
            for d in range(_DEG):
                alpha_v[p * _DEG + d, :] = ps[d] * rinv

    def accum_part(tt, cur):
        nb = node_base(tt)
        for p in range(_NB // 2):
            for j in range(2):
                for half in range(2):
                    zero = jnp.zeros((16,), _f32)

                    @plsc.parallel_loop(0, _DEG, unroll=2,
                                        carry=(zero,) * 32)
                    def accs(d, accs_in, _p=p, _j=j, _h=half):
                        r = _p * 2 * _DEG + 2 * d + _j
                        xs = [xg_v[cur, r, pl.ds(_h * 64 + l * 16, 16)]
                              for l in range(4)]
                        av = alpha_v[_p * _DEG + d, :]
                        new = list(accs_in)
                        for k in range(_H):
                            al = av[_j * 8 + k]
                            for l in range(4):
                                new[k * 4 + l] = new[k * 4 + l] + al * xs[l]
                        return tuple(new)
                    srow = (p * 2 + j) * (_H * _D)
                    for k in range(_H):
                        for l in range(4):
                            s_v[pl.ds(srow + k * _D + half * 64 + l * 16,
                                      16)] = accs[k * 4 + l]
        pltpu.sync_copy(s_v, out.at[pl.ds(nb * (_H * _D), _NB * _H * _D)])

    def outer(o, carry):
        for b2 in range(2):
            tt = o * 2 + b2
            cur, nxt = b2, 1 - b2
            wait_idx()
            fire_gather(nxt)
            softmax_part(cur)
            wait_gather(cur)
            fire_idx(jnp.minimum(tt + 2, _TBLK - 1), cur)
            accum_part(tt, cur)
        return carry

    lax.fori_loop(0, _TBLK // 2, outer, 0)
    wait_idx()
    wait_gather(0)


_sc_agg_built = None


def _sc_agg(table, esrc_flat, edst_flat, nbrs_flat):
    global _sc_agg_built
    if _sc_agg_built is None:
        mesh = plsc.VectorSubcoreMesh(core_axis_name="c", subcore_axis_name="s")
        _sc_agg_built = pl.kernel(
            _sc_body,
            out_type=jax.ShapeDtypeStruct((_NP * _H * _D,), _f32),
            mesh=mesh,
            compiler_params=pltpu.CompilerParams(needs_layout_passes=False),
            scratch_types=[
                pltpu.VMEM((2, _ROWS), jnp.int32),
                pltpu.VMEM((2 * _NB * _H,), _f32),
                pltpu.VMEM((_NP * _H,), _f32),
                pltpu.VMEM((2, _ROWS, _D), _f32),
                pltpu.VMEM((_NB * _H * _D,), _f32),
                pltpu.VMEM(((_NB // 2) * _DEG, 16), _f32),
                pltpu.SemaphoreType.DMA,
                pltpu.SemaphoreType.DMA,
                pltpu.SemaphoreType.DMA,
                pltpu.SemaphoreType.DMA,
            ],
        )
    return _sc_agg_built(table, esrc_flat, edst_flat, nbrs_flat)


def kernel(node_features, neighbours, W1, a1, W2, a2, ln_gamma, ln_beta,
           lin1_W, lin1_b, lin2_W, lin2_b, lin3_W, lin3_b):
    pad = _NP - _N
    x = jnp.pad(node_features, ((0, pad), (0, 0)))
    nbrs = jnp.pad(neighbours.astype(jnp.int32), ((0, pad), (0, 0)))
    nbp = nbrs.reshape(_NP // 2, 2, _DEG).transpose(0, 2, 1).reshape(-1)

    b1s = jnp.einsum('khc,kh->ck', W1, a1[:, :_D])
    b1d = jnp.einsum('khc,kh->ck', W1, a1[:, _D:])
    b2s = jnp.einsum('khc,kh->ck', W2, a2[:, :_D])
    b2d = jnp.einsum('khc,kh->ck', W2, a2[:, _D:])
    w1c = W1.transpose(0, 2, 1).reshape(_H * _D, _D)
    w2c = W2.transpose(0, 2, 1).reshape(_H * _D, _D)

    es1, ed1 = _tc_logits1(x, b1s, b1d)
    s1 = _sc_agg(x, es1.reshape(-1), ed1.reshape(-1), nbp).reshape(_NP, -1)
    e1, es2, ed2 = _tc_fuse2(s1, w1c, b2s, b2d)
    s2 = _sc_agg(e1, es2.reshape(-1), ed2.reshape(-1), nbp).reshape(_NP, -1)
    out = _tc_head(s2, w2c,
                   ln_gamma.reshape(1, -1), ln_beta.reshape(1, -1),
                   lin1_W.T, lin1_b.reshape(1, -1),
                   lin2_W.T, lin2_b.reshape(1, -1),
                   lin3_W.T, lin3_b.reshape(1, -1))
    return out.reshape(16)


# speedup vs baseline: 38.6773x; 1.0990x over previous
import jax
import jax.numpy as jnp
from jax import lax
from jax.experimental import pallas as pl
from jax.experimental.pallas import tpu as pltpu
from jax.experimental.pallas import tpu_sc as plsc

_N = 10000
_NP = 10240
_DEG = 32
_D = 128
_H = 8
_NB = 4
_ROWS = _NB * _DEG
_NTILES = 32
_TBLK = _NP // (_NB * _NTILES)
_TBLK_C0 = 96
_TBLK_C1 = 64
_f32 = jnp.float32


def _logits_body(x_ref, bs_ref, bd_ref, es_ref, ed_ref):
    xb = x_ref[:]
    es_ref[:] = jnp.dot(xb, bs_ref[:], preferred_element_type=_f32)
    ed_ref[:] = jnp.dot(xb, bd_ref[:], preferred_element_type=_f32)


def _tc_logits1(x, bsrc, bdst):
    R = 2048
    return pl.pallas_call(
        _logits_body,
        grid=(_NP // R,),
        in_specs=[
            pl.BlockSpec((R, _D), lambda i: (i, 0)),
            pl.BlockSpec((_D, _H), lambda i: (0, 0)),
            pl.BlockSpec((_D, _H), lambda i: (0, 0)),
        ],
        out_specs=[
            pl.BlockSpec((R, _H), lambda i: (i, 0)),
            pl.BlockSpec((R, _H), lambda i: (i, 0)),
        ],
        out_shape=[
            jax.ShapeDtypeStruct((_NP, _H), _f32),
            jax.ShapeDtypeStruct((_NP, _H), _f32),
        ],
    )(x, bsrc, bdst)


def _fuse2_body(s_ref, wc_ref, bs_ref, bd_ref, e1_ref, es_ref, ed_ref):
    agg = jnp.dot(s_ref[:], wc_ref[:], preferred_element_type=_f32)
    z = agg * (1.0 / _H)
    e1 = jnp.where(z > 0, z, jnp.exp(z) - 1.0)
    e1_ref[:] = e1
    es_ref[:] = jnp.dot(e1, bs_ref[:], preferred_element_type=_f32)
    ed_ref[:] = jnp.dot(e1, bd_ref[:], preferred_element_type=_f32)


def _tc_fuse2(s, wcat, bsrc, bdst):
    R = 1024
    return pl.pallas_call(
        _fuse2_body,
        grid=(_NP // R,),
        in_specs=[
            pl.BlockSpec((R, _H * _D), lambda i: (i, 0)),
            pl.BlockSpec((_H * _D, _D), lambda i: (0, 0)),
            pl.BlockSpec((_D, _H), lambda i: (0, 0)),
            pl.BlockSpec((_D, _H), lambda i: (0, 0)),
        ],
        out_specs=[
            pl.BlockSpec((R, _D), lambda i: (i, 0)),
            pl.BlockSpec((R, _H), lambda i: (i, 0)),
            pl.BlockSpec((R, _H), lambda i: (i, 0)),
        ],
        out_shape=[
            jax.ShapeDtypeStruct((_NP, _D), _f32),
            jax.ShapeDtypeStruct((_NP, _H), _f32),
            jax.ShapeDtypeStruct((_NP, _H), _f32),
        ],
    )(s, wcat, bsrc, bdst)


def _head_body(s_ref, wc_ref, gam_ref, bet_ref, w1_ref, b1_ref, w2_ref,
               b2_ref, w3_ref, b3_ref, out_ref, acc_ref):
    i = pl.program_id(0)
    R = s_ref.shape[0]
    agg = jnp.dot(s_ref[:], wc_ref[:], preferred_element_type=_f32)
    z = agg * (1.0 / _H)
    z = jnp.where(z > 0, z, jnp.exp(z) - 1.0)
    rowid = i * R + lax.broadcasted_iota(jnp.int32, (R, _D), 0)
    z = jnp.where(rowid < _N, z, 0.0)
    part = jnp.sum(z, axis=0, keepdims=True)

    @pl.when(i == 0)
    def _():
        acc_ref[:] = part

    @pl.when(i > 0)
    def _():
        acc_ref[:] = acc_ref[:] + part

    @pl.when(i == pl.num_programs(0) - 1)
    def _():
        g = acc_ref[:] * (1.0 / _N)
        mu = jnp.mean(g, axis=1, keepdims=True)
        var = jnp.mean((g - mu) ** 2, axis=1, keepdims=True)
        y = (g - mu) * lax.rsqrt(var + 1e-5) * gam_ref[:] + bet_ref[:]
        h1 = jnp.dot(y, w1_ref[:], preferred_element_type=_f32) + b1_ref[:]
        h1 = jnp.where(h1 > 0, h1, 0.01 * h1)
        h2 = jnp.dot(h1, w2_ref[:], preferred_element_type=_f32) + b2_ref[:]
        h2 = jnp.where(h2 > 0, h2, 0.01 * h2)
        h3 = jnp.dot(h2, w3_ref[:], preferred_element_type=_f32) + b3_ref[:]
        out_ref[:] = jnp.maximum(h3, 0.0)


def _tc_head(s, wcat, gam, bet, w1t, b1, w2t, b2, w3t, b3):
    R = 1024
    full = lambda shape: pl.BlockSpec(shape, lambda i: (0,) * len(shape))
    return pl.pallas_call(
        _head_body,
        grid=(_NP // R,),
        in_specs=[
            pl.BlockSpec((R, _H * _D), lambda i: (i, 0)),
            full((_H * _D, _D)),
            full((1, _D)), full((1, _D)),
            full((_D, 64)), full((1, 64)),
            full((64, 16)), full((1, 16)),
            full((16, 16)), full((1, 16)),
        ],
        out_specs=full((1, 16)),
        out_shape=jax.ShapeDtypeStruct((1, 16), _f32),
        scratch_shapes=[pltpu.VMEM((1, _D), _f32)],
    )(s, wcat, gam, bet, w1t, b1, w2t, b2, w3t, b3)


def _sc_body(table, esrc, edst, nbrs, out, idx_v, es_v, edst_v, xg_v, s_v,
             alpha_v, sem_i, sem_e, sem_g0, sem_g1):
    sem_g = (sem_g0, sem_g1)
    cid = lax.axis_index("c")
    sid = lax.axis_index("s")
    my_nblk = jnp.where(cid == 0, _TBLK_C0, _TBLK_C1)
    base = jnp.where(cid == 0, sid * _TBLK_C0,
                     16 * _TBLK_C0 + sid * _TBLK_C1)
    lanes_lo = lax.iota(jnp.int32, 16) < 8
    col8 = lax.rem(lax.iota(jnp.int32, 16), 8)

    def node_base(t):
        return (base + t) * _NB

    def fire_idx(t, b):
        nb = node_base(t)
        pltpu.make_async_copy(
            nbrs.at[pl.ds(nb * _DEG, _ROWS)], idx_v.at[b], sem_i).start()
        pltpu.make_async_copy(
            esrc.at[pl.ds(nb * _H, _NB * _H)],
            es_v.at[pl.ds(b * _NB * _H, _NB * _H)], sem_e).start()

    def wait_idx():
        pltpu.make_async_copy(
            nbrs.at[pl.ds(0, _ROWS)], idx_v.at[0], sem_i).wait()
        pltpu.make_async_copy(
            esrc.at[pl.ds(0, _NB * _H)],
            es_v.at[pl.ds(0, _NB * _H)], sem_e).wait()

    def fire_gather(b):
        pltpu.make_async_copy(
            table.at[idx_v.at[b]], xg_v.at[b], sem_g[b]).start()

    def wait_gather(b):
        pltpu.make_async_copy(
            table.at[idx_v.at[b]], xg_v.at[b], sem_g[b]).wait()

    pltpu.sync_copy(edst, edst_v)

    fire_idx(0, 0)
    wait_idx()
    fire_gather(0)
    fire_idx(jnp.minimum(1, my_nblk - 1), 1)

    def softmax_part(cur):
        for p in range(_NB // 2):
            es = es_v[pl.ds(cur * _NB * _H + p * 16, 16)]
            nchunk = [idx_v[cur, pl.ds(p * 2 * _DEG + q * 16, 16)]
                      for q in range(4)]
            evs = []
            for d in range(_DEG):
                v = nchunk[d // 8]
                n0 = v[(2 * d) % 16]
                n1 = v[(2 * d + 1) % 16]
                row = jnp.where(lanes_lo, n0, n1)
                ed = plsc.load_gather(edst_v, [row * 8 + col8])
                e = es + ed
                evs.append(jnp.where(e > 0, e, 0.01 * e))
            m = evs[0]
            for d in range(1, _DEG):
                m = jnp.maximum(m, evs[d])
            ps = [jnp.exp(e - m) for e in evs]
            ssum = ps[0]
            for d in range(1, _DEG):
                ssum = ssum + ps[d]
            rinv = 1.0 / ssum
            for d in range(_DEG):
                alpha_v[p * _DEG + d, :] = ps[d] * rinv

    def accum_part(tt, cur):
        nb = node_base(tt)
        for p in range(_NB // 2):
            for j in range(2):
                for half in range(2):
                    zero = jnp.zeros((16,), _f32)

                    @plsc.parallel_loop(0, _DEG, unroll=2,
                                        carry=(zero,) * 32)
                    def accs(d, accs_in, _p=p, _j=j, _h=half):
                        r = _p * 2 * _DEG + 2 * d + _j
                        xs = [xg_v[cur, r, pl.ds(_h * 64 + l * 16, 16)]
                              for l in range(4)]
                        av = alpha_v[_p * _DEG + d, :]
                        new = list(accs_in)
                        for k in range(_H):
                            al = av[_j * 8 + k]
                            for l in range(4):
                                new[k * 4 + l] = new[k * 4 + l] + al * xs[l]
                        return tuple(new)
                    srow = (p * 2 + j) * (_H * _D)
                    for k in range(_H):
                        for l in range(4):
                            s_v[pl.ds(srow + k * _D + half * 64 + l * 16,
                                      16)] = accs[k * 4 + l]
        pltpu.sync_copy(s_v, out.at[pl.ds(nb * (_H * _D), _NB * _H * _D)])

    def outer(o, carry):
        for b2 in range(2):
            tt = o * 2 + b2
            cur, nxt = b2, 1 - b2
            wait_idx()
            fire_gather(nxt)
            softmax_part(cur)
            wait_gather(cur)
            fire_idx(jnp.minimum(tt + 2, my_nblk - 1), cur)
            accum_part(tt, cur)
        return carry

    lax.fori_loop(0, my_nblk // 2, outer, 0)
    wait_idx()
    wait_gather(0)


_sc_agg_built = None


def _sc_agg(table, esrc_flat, edst_flat, nbrs_flat):
    global _sc_agg_built
    if _sc_agg_built is None:
        mesh = plsc.VectorSubcoreMesh(core_axis_name="c", subcore_axis_name="s")
        _sc_agg_built = pl.kernel(
            _sc_body,
            out_type=jax.ShapeDtypeStruct((_NP * _H * _D,), _f32),
            mesh=mesh,
            compiler_params=pltpu.CompilerParams(needs_layout_passes=False),
            scratch_types=[
                pltpu.VMEM((2, _ROWS), jnp.int32),
                pltpu.VMEM((2 * _NB * _H,), _f32),
                pltpu.VMEM((_NP * _H,), _f32),
                pltpu.VMEM((2, _ROWS, _D), _f32),
                pltpu.VMEM((_NB * _H * _D,), _f32),
                pltpu.VMEM(((_NB // 2) * _DEG, 16), _f32),
                pltpu.SemaphoreType.DMA,
                pltpu.SemaphoreType.DMA,
                pltpu.SemaphoreType.DMA,
                pltpu.SemaphoreType.DMA,
            ],
        )
    return _sc_agg_built(table, esrc_flat, edst_flat, nbrs_flat)


def kernel(node_features, neighbours, W1, a1, W2, a2, ln_gamma, ln_beta,
           lin1_W, lin1_b, lin2_W, lin2_b, lin3_W, lin3_b):
    pad = _NP - _N
    x = jnp.pad(node_features, ((0, pad), (0, 0)))
    nbrs = jnp.pad(neighbours.astype(jnp.int32), ((0, pad), (0, 0)))
    nbp = nbrs.reshape(_NP // 2, 2, _DEG).transpose(0, 2, 1).reshape(-1)

    b1s = jnp.einsum('khc,kh->ck', W1, a1[:, :_D])
    b1d = jnp.einsum('khc,kh->ck', W1, a1[:, _D:])
    b2s = jnp.einsum('khc,kh->ck', W2, a2[:, :_D])
    b2d = jnp.einsum('khc,kh->ck', W2, a2[:, _D:])
    w1c = W1.transpose(0, 2, 1).reshape(_H * _D, _D)
    w2c = W2.transpose(0, 2, 1).reshape(_H * _D, _D)

    es1, ed1 = _tc_logits1(x, b1s, b1d)
    s1 = _sc_agg(x, es1.reshape(-1), ed1.reshape(-1), nbp).reshape(_NP, -1)
    e1, es2, ed2 = _tc_fuse2(s1, w1c, b2s, b2d)
    s2 = _sc_agg(e1, es2.reshape(-1), ed2.reshape(-1), nbp).reshape(_NP, -1)
    out = _tc_head(s2, w2c,
                   ln_gamma.reshape(1, -1), ln_beta.reshape(1, -1),
                   lin1_W.T, lin1_b.reshape(1, -1),
                   lin2_W.T, lin2_b.reshape(1, -1),
                   lin3_W.T, lin3_b.reshape(1, -1))
    return out.reshape(16)
